# Initial kernel scaffold; baseline (speedup 1.0000x reference)
import jax
import jax.numpy as jnp
from jax.experimental import pallas as pl


def kernel(pred0, pred1, pred2, targets):
    raise NotImplementedError("write your pallas kernel here")


# speedup vs baseline: 53.8005x; 53.8005x over previous
import functools

import numpy as np
import jax
import jax.numpy as jnp
from jax import lax
from jax.experimental import pallas as pl
from jax.experimental.pallas import tpu as pltpu
from jax.experimental.pallas import tpu_sc as plsc

_IMG = 640.0
_STRIDES = (8, 16, 32)
_B = 16
_G = 32
_NC = 96
_TASKS = 48


def _sc_body(gtb_hbm, p0, p1, p2, out_hbm, bmap, gtv, stage, idxv, sem):
    wid = lax.axis_index("s") * 2 + lax.axis_index("c")

    def run_task(task):
        level = task // 16
        b = task % 16
        W = jnp.where(level == 0, 80, jnp.where(level == 1, 40, 20)).astype(jnp.int32)
        Wf = W.astype(jnp.float32)
        HW = W * W

        pltpu.sync_copy(gtb_hbm.at[b], gtv)

        cands = [None] * 6
        for h in range(2):
            sl = pl.ds(h * 16, 16)
            cx = gtv[0, sl]
            cy = gtv[1, sl]
            gx = cx * Wf
            gy = cy * Wf
            col = jnp.clip(gx.astype(jnp.int32), 0, W - 1)
            row = jnp.clip(gy.astype(jnp.int32), 0, W - 1)
            offx = gx - col.astype(jnp.float32)
            offy = gy - row.astype(jnp.float32)
            ltx = offx < 0.5
            lty = offy < 0.5
            nx = jnp.where(ltx, col - 1, col + 1)
            vx = jnp.where(ltx, col > 0, col < W - 1)
            ny = jnp.where(lty, row - 1, row + 1)
            vy = jnp.where(lty, row > 0, row < W - 1)
            g = lax.iota(jnp.int32, 16) + (h * 16)
            neg = jnp.full((16,), -1, jnp.int32)
            cands[0 + h] = (row * W + col, g)
            cands[2 + h] = (row * W + jnp.clip(nx, 0, W - 1), jnp.where(vx, g, neg))
            cands[4 + h] = (jnp.clip(ny, 0, W - 1) * W + col, jnp.where(vy, g, neg))

        zeros16 = jnp.zeros((16,), jnp.int32)
        one16 = jnp.ones((16,), jnp.int32)
        for idx, _ in cands:
            plsc.store_scatter(bmap, [idx], zeros16)
        for idx, val in cands:
            vs = jnp.maximum(val, 0)
            plsc.addupdate_scatter(bmap, [idx], one16 << vs, mask=val >= 0)
        base = (b * 5) * HW
        for slot, (idx, val) in enumerate(cands):
            got = plsc.load_gather(bmap, [idx])
            vs = jnp.maximum(val, 0)
            above = jnp.full((16,), -2, jnp.int32) << vs
            win = (val >= 0) & ((got & above) == 0)
            cs = pl.ds(slot * 16, 16)
            stage[0, cs] = win.astype(jnp.float32)
            stage[6, cs] = idx.astype(jnp.float32)
            for c in range(5):
                idxv[c, cs] = idx + (base + c * HW)

        def gather_from(ph):
            def _go():
                hs = [
                    pltpu.async_copy(ph.at[idxv.at[c]], stage.at[1 + c], sem)
                    for c in range(5)
                ]
                for hh in hs:
                    hh.wait()
            return _go

        pl.when(level == 0)(gather_from(p0))
        pl.when(level == 1)(gather_from(p1))
        pl.when(level == 2)(gather_from(p2))

        pltpu.sync_copy(stage, out_hbm.at[task])

    run_task(wid)

    @pl.when(wid < 16)
    def _():
        run_task(wid + 32)


@jax.jit
def _sc_assign(gtb_t, p0f, p1f, p2f):
    mesh = plsc.VectorSubcoreMesh(core_axis_name="c", subcore_axis_name="s")
    return pl.kernel(
        _sc_body,
        out_type=jax.ShapeDtypeStruct((_TASKS, 7, _NC), jnp.float32),
        mesh=mesh,
        scratch_types=[
            pltpu.VMEM((6400,), jnp.int32),
            pltpu.VMEM((4, _G), jnp.float32),
            pltpu.VMEM((7, _NC), jnp.float32),
            pltpu.VMEM((5, _NC), jnp.int32),
            pltpu.SemaphoreType.DMA,
        ],
        compiler_params=pltpu.CompilerParams(needs_layout_passes=False),
    )(gtb_t, p0f, p1f, p2f)


def _tc_body(p0, p1, p2, tg, sc, out):
    soft = jnp.float32(0.0)
    for p in (p0, p1, p2):
        l = p[:, 0]
        soft = soft + jnp.sum(jnp.maximum(l, 0.0) + jnp.log1p(jnp.exp(-jnp.abs(l))))

    gtb = tg[:, :, 1:5]
    t_cx = jnp.concatenate([gtb[..., 0]] * 3, axis=1)
    t_cy = jnp.concatenate([gtb[..., 1]] * 3, axis=1)
    t_w = jnp.concatenate([gtb[..., 2]] * 3, axis=1)
    t_h = jnp.concatenate([gtb[..., 3]] * 3, axis=1)
    tx1 = t_cx - t_w * 0.5
    ty1 = t_cy - t_h * 0.5
    tx2 = t_cx + t_w * 0.5
    ty2 = t_cy + t_h * 0.5

    eps = 1e-7
    box_total = jnp.float32(0.0)
    items = jnp.float32(0.0)
    tpos = jnp.float32(0.0)
    corr = jnp.float32(0.0)
    for lvl, stride in enumerate(_STRIDES):
        Wl = int(_IMG) // stride
        s = float(stride) / _IMG
        scl = sc[lvl]
        win = scl[:, 0, :]
        lg = scl[:, 1, :]
        idxi = scl[:, 6, :].astype(jnp.int32)
        r = (idxi // Wl).astype(jnp.float32)
        c = (idxi % Wl).astype(jnp.float32)
        p_cx = (jax.nn.sigmoid(scl[:, 2, :]) * 2.0 - 0.5 + c) * s
        p_cy = (jax.nn.sigmoid(scl[:, 3, :]) * 2.0 - 0.5 + r) * s
        p_w = jnp.exp(jnp.clip(scl[:, 4, :], -5.0, 5.0)) * s
        p_h = jnp.exp(jnp.clip(scl[:, 5, :], -5.0, 5.0)) * s
        px1 = p_cx - p_w * 0.5
        py1 = p_cy - p_h * 0.5
        px2 = p_cx + p_w * 0.5
        py2 = p_cy + p_h * 0.5

        iw = jnp.clip(jnp.minimum(px2, tx2) - jnp.maximum(px1, tx1), 0.0, None)
        ih = jnp.clip(jnp.minimum(py2, ty2) - jnp.maximum(py1, ty1), 0.0, None)
        inter = iw * ih
        union = p_w * p_h + t_w * t_h - inter + eps
        iou = inter / union
        cw = jnp.maximum(px2, tx2) - jnp.minimum(px1, tx1)
        ch = jnp.maximum(py2, ty2) - jnp.minimum(py1, ty1)
        c2 = cw * cw + ch * ch + eps
        rho2 = ((tx1 + tx2 - px1 - px2) ** 2 + (ty1 + ty2 - py1 - py2) ** 2) / 4.0
        v = (4.0 / (np.pi ** 2)) * jnp.square(
            jnp.arctan2(t_w, t_h + eps) - jnp.arctan2(p_w, p_h + eps))
        alpha = v / (v - iou + 1.0 + eps)
        cl = jnp.clip(1.0 - (iou - (rho2 / c2 + alpha * v)), 0.0, None)

        ap = jnp.clip(p_w * p_h, 1e-7, None)
        at = jnp.clip(t_w * t_h, 1e-7, None)
        iou2 = jnp.clip(inter / (ap + at - inter + 1e-7), 0.0, 1.0)

        npos_b = jnp.sum(win, axis=1)
        box_sum_b = jnp.sum(win * cl, axis=1)
        corr = corr + jnp.sum(win * lg * iou2)
        box_total = box_total + jnp.sum(
            jnp.where(npos_b > 0, box_sum_b / jnp.maximum(npos_b, 1.0), 0.0))
        items = items + jnp.sum((npos_b > 0).astype(jnp.float32))
        tpos = tpos + jnp.sum(npos_b)

    total = (soft - corr) / jnp.maximum(1.0, tpos) + 5.0 * box_total / jnp.maximum(1.0, items)
    out[0, 0] = total


def _tc_finish(pred0, pred1, pred2, targets, sc, interpret=False):
    zmap = lambda i: (0, 0, 0, 0)
    return pl.pallas_call(
        _tc_body,
        grid=(1,),
        out_shape=jax.ShapeDtypeStruct((1, 1), jnp.float32),
        in_specs=[
            pl.BlockSpec((_B, 1, 80, 80), zmap),
            pl.BlockSpec((_B, 1, 40, 40), zmap),
            pl.BlockSpec((_B, 1, 20, 20), zmap),
            pl.BlockSpec(targets.shape, lambda i: (0, 0, 0)),
            pl.BlockSpec(sc.shape, lambda i: (0, 0, 0, 0)),
        ],
        out_specs=pl.BlockSpec((1, 1), lambda i: (0, 0), memory_space=pltpu.SMEM),
        interpret=interpret,
    )(pred0, pred1, pred2, targets, sc)


def kernel(pred0, pred1, pred2, targets):
    gtb_t = jnp.transpose(targets[:, :, 1:5], (0, 2, 1))
    p0f = pred0.reshape(-1)
    p1f = pred1.reshape(-1)
    p2f = pred2.reshape(-1)
    scout = _sc_assign(gtb_t, p0f, p1f, p2f)
    sc = scout.reshape(3, _B, 7, _NC)
    res = _tc_finish(pred0, pred1, pred2, targets, sc)
    return res[0, 0]
